# baseline (speedup 1.0000x reference)
import jax
import jax.numpy as jnp
from jax import lax
from jax.experimental import pallas as pl

T = 128
D = 1024
FF = 512
E = 64
K = 8
EB = 4
NS = E // EB


def _moe_body(idx_ref, wts_ref, x_ref, w1_ref, w3_ref, w2_ref, out_ref):
    s = pl.program_id(0)
    x = x_ref[...]
    idx = idx_ref[...]
    wts = wts_ref[...]

    acc = jnp.zeros((T, D), jnp.float32)
    for j in range(EB):
        e = s * EB + j
        w1 = w1_ref[j]
        w3 = w3_ref[j]
        w2 = w2_ref[j]
        g = lax.dot_general(x, w1, (((1,), (1,)), ((), ())),
                            preferred_element_type=jnp.float32)
        u = lax.dot_general(x, w3, (((1,), (1,)), ((), ())),
                            preferred_element_type=jnp.float32)
        h = jax.nn.gelu(g, approximate=True) * u
        y = lax.dot_general(h, w2, (((1,), (1,)), ((), ())),
                            preferred_element_type=jnp.float32)
        coef = jnp.sum(jnp.where(idx == e, wts, 0.0), axis=1)
        acc = acc + coef[:, None] * y

    @pl.when(s == 0)
    def _():
        out_ref[...] = acc

    @pl.when(s != 0)
    def _():
        out_ref[...] += acc


def kernel(hidden_states, top_k_index, top_k_weights, w1_weight, w2_weight, w3_weight):
    top_k_index = top_k_index.astype(jnp.int32)
    return pl.pallas_call(
        _moe_body,
        grid=(NS,),
        in_specs=[
            pl.BlockSpec((T, K), lambda s: (0, 0)),
            pl.BlockSpec((T, K), lambda s: (0, 0)),
            pl.BlockSpec((T, D), lambda s: (0, 0)),
            pl.BlockSpec((EB, FF, D), lambda s: (s, 0, 0)),
            pl.BlockSpec((EB, FF, D), lambda s: (s, 0, 0)),
            pl.BlockSpec((EB, D, FF), lambda s: (s, 0, 0)),
        ],
        out_specs=pl.BlockSpec((T, D), lambda s: (0, 0)),
        out_shape=jax.ShapeDtypeStruct((T, D), jnp.float32),
    )(top_k_index, top_k_weights, hidden_states, w1_weight, w3_weight, w2_weight)
